# baseline (speedup 1.0000x reference)
import functools

import jax
import jax.numpy as jnp
from jax import lax
from jax.experimental import pallas as pl
from jax.experimental.pallas import tpu as pltpu
from jax.experimental.pallas import tpu_sc as plsc

N = 10000
E = 320000
D = 128
BN_EPS = 1e-5

NC = 2
NS = 16
NW = NC * NS
EPW = E // NW
CH = 125
NCH = EPW // CH
SW = 624
TAIL = N - NS * SW
ZR = 16

_mesh = plsc.VectorSubcoreMesh(core_axis_name="c", subcore_axis_name="s")


@functools.partial(
    pl.kernel,
    out_type=jax.ShapeDtypeStruct((NC, N, D), jnp.float32),
    mesh=_mesh,
    scratch_types=[
        pltpu.VMEM((2, CH), jnp.int32),
        pltpu.VMEM((2, CH), jnp.int32),
        pltpu.VMEM((CH, D), jnp.float32),
        pltpu.VMEM((CH, D), jnp.float32),
        pltpu.VMEM((ZR, D), jnp.float32),
        pltpu.VMEM_SHARED((N, D), jnp.float32),
        pltpu.SemaphoreType.DMA,
        pltpu.SemaphoreType.DMA,
        pltpu.SemaphoreType.DMA,
        pltpu.SemaphoreType.DMA,
    ],
)
def _sc_agg(x_hbm, eidx_hbm, out_hbm, i0, i1, g0, g1, zbuf, acc,
            isem0, isem1, gsem0, gsem1):
    c = lax.axis_index("c")
    s = lax.axis_index("s")
    wid = s * NC + c
    ib = (i0, i1)
    gb = (g0, g1)
    isem = (isem0, isem1)
    gsem = (gsem0, gsem1)

    @pl.when(c == 0)
    def _init_x():
        pltpu.sync_copy(x_hbm.at[pl.ds(s * SW, SW)], acc.at[pl.ds(s * SW, SW)])

        @pl.when(s == NS - 1)
        def _init_x_tail():
            pltpu.sync_copy(x_hbm.at[pl.ds(NS * SW, TAIL)],
                            acc.at[pl.ds(NS * SW, TAIL)])

    @pl.when(c == 1)
    def _init_zero():
        def _zrow(i, _):
            for j in range(D // 16):
                zbuf[i, pl.ds(j * 16, 16)] = jnp.zeros((16,), jnp.float32)
            return 0

        lax.fori_loop(0, ZR, _zrow, 0)
        for r in range(SW // ZR):
            pltpu.sync_copy(zbuf, acc.at[pl.ds(s * SW + r * ZR, ZR)])

        @pl.when(s == NS - 1)
        def _zero_tail():
            pltpu.sync_copy(zbuf, acc.at[pl.ds(NS * SW, TAIL)])

    plsc.subcore_barrier()

    pltpu.async_copy(eidx_hbm.at[wid, 0], i0, isem0).wait()
    pltpu.async_copy(eidx_hbm.at[wid, 1], i1, isem1)
    pltpu.async_copy(x_hbm.at[i0.at[0]], g0, gsem0)

    def _half_step(j, b, fire_idx):
        nb = 1 - b
        pltpu.make_async_copy(x_hbm.at[ib[b].at[0]], gb[b], gsem[b]).wait()
        pltpu.make_async_copy(eidx_hbm.at[wid, j], ib[nb], isem[nb]).wait()
        pltpu.async_copy(x_hbm.at[ib[nb].at[0]], gb[nb], gsem[nb])
        pltpu.sync_copy(gb[b], acc.at[ib[b].at[1]], add=True)
        if fire_idx:
            pltpu.async_copy(eidx_hbm.at[wid, j + 2], ib[b], isem[b])

    def _pair(i, _):
        j = i * 2
        _half_step(j, 0, True)
        _half_step(j + 1, 1, True)
        return 0

    lax.fori_loop(0, NCH // 2 - 1, _pair, 0)
    _half_step(NCH - 2, 0, False)
    pltpu.make_async_copy(x_hbm.at[i1.at[0]], g1, gsem1).wait()
    pltpu.sync_copy(g1, acc.at[i1.at[1]], add=True)

    plsc.subcore_barrier()
    pltpu.sync_copy(acc.at[pl.ds(s * SW, SW)],
                    out_hbm.at[c].at[pl.ds(s * SW, SW)])

    @pl.when(s == NS - 1)
    def _write_tail():
        pltpu.sync_copy(acc.at[pl.ds(NS * SW, TAIL)],
                        out_hbm.at[c].at[pl.ds(NS * SW, TAIL)])


def _mlp_body(p_ref, w1_ref, b1_ref, g_ref, be_ref, w2_ref, b2_ref, o_ref):
    xu = p_ref[0] + p_ref[1]
    h = lax.dot_general(xu, w1_ref[...], (((1,), (1,)), ((), ())),
                        precision=lax.Precision.HIGHEST,
                        preferred_element_type=jnp.float32)
    h = h + b1_ref[...]
    mean = jnp.mean(h, axis=0, keepdims=True)
    d = h - mean
    var = jnp.mean(d * d, axis=0, keepdims=True)
    h = g_ref[...] * d * lax.rsqrt(var + BN_EPS) + be_ref[...]
    h = jnp.maximum(h, 0.0)
    o_ref[...] = lax.dot_general(h, w2_ref[...], (((1,), (1,)), ((), ())),
                                 precision=lax.Precision.HIGHEST,
                                 preferred_element_type=jnp.float32) + b2_ref[...]


@jax.jit
def kernel(x, edge_index, W1, b1, gamma, beta, W2, b2):
    ei = edge_index.astype(jnp.int32)
    eidx = jnp.stack(
        [ei[1].reshape(NW, NCH, CH), ei[0].reshape(NW, NCH, CH)], axis=2)
    parts = _sc_agg(x, eidx)
    return pl.pallas_call(
        _mlp_body,
        out_shape=jax.ShapeDtypeStruct((N, D), jnp.float32),
    )(parts, W1, b1.reshape(1, D), gamma.reshape(1, D),
      beta.reshape(1, D), W2, b2.reshape(1, D))


# speedup vs baseline: 1.0744x; 1.0744x over previous
import functools

import jax
import jax.numpy as jnp
from jax import lax
from jax.experimental import pallas as pl
from jax.experimental.pallas import tpu as pltpu
from jax.experimental.pallas import tpu_sc as plsc

N = 10000
E = 320000
D = 128
BN_EPS = 1e-5

NC = 2
NS = 16
NW = NC * NS
EPW = E // NW
CH = 125
NCH = EPW // CH
SW = 624
TAIL = N - NS * SW
ZR = 16

_mesh = plsc.VectorSubcoreMesh(core_axis_name="c", subcore_axis_name="s")


@functools.partial(
    pl.kernel,
    out_type=jax.ShapeDtypeStruct((NC, N, D), jnp.float32),
    mesh=_mesh,
    scratch_types=[
        pltpu.VMEM((2, CH), jnp.int32),
        pltpu.VMEM((2, CH), jnp.int32),
        pltpu.VMEM((CH, D), jnp.float32),
        pltpu.VMEM((CH, D), jnp.float32),
        pltpu.VMEM((ZR, D), jnp.float32),
        pltpu.VMEM_SHARED((N, D), jnp.float32),
        pltpu.SemaphoreType.DMA,
        pltpu.SemaphoreType.DMA,
        pltpu.SemaphoreType.DMA,
        pltpu.SemaphoreType.DMA,
    ],
)
def _sc_agg(x_hbm, eidx_hbm, out_hbm, i0, i1, g0, g1, zbuf, acc,
            isem0, isem1, gsem0, gsem1):
    c = lax.axis_index("c")
    s = lax.axis_index("s")
    wid = s * NC + c
    ib = (i0, i1)
    gb = (g0, g1)
    isem = (isem0, isem1)
    gsem = (gsem0, gsem1)

    @pl.when(c == 0)
    def _init_x():
        pltpu.sync_copy(x_hbm.at[pl.ds(s * SW, SW)], acc.at[pl.ds(s * SW, SW)])

        @pl.when(s == NS - 1)
        def _init_x_tail():
            pltpu.sync_copy(x_hbm.at[pl.ds(NS * SW, TAIL)],
                            acc.at[pl.ds(NS * SW, TAIL)])

    @pl.when(c == 1)
    def _init_zero():
        def _zrow(i, _):
            for j in range(D // 16):
                zbuf[i, pl.ds(j * 16, 16)] = jnp.zeros((16,), jnp.float32)
            return 0

        lax.fori_loop(0, ZR, _zrow, 0)
        for r in range(SW // ZR):
            pltpu.sync_copy(zbuf, acc.at[pl.ds(s * SW + r * ZR, ZR)])

        @pl.when(s == NS - 1)
        def _zero_tail():
            pltpu.sync_copy(zbuf, acc.at[pl.ds(NS * SW, TAIL)])

    plsc.subcore_barrier()

    pltpu.async_copy(eidx_hbm.at[wid, 0], i0, isem0).wait()
    pltpu.async_copy(eidx_hbm.at[wid, 1], i1, isem1)
    pltpu.async_copy(x_hbm.at[i0.at[0]], g0, gsem0)

    def _half_step(j, b, fire_idx):
        nb = 1 - b
        pltpu.make_async_copy(x_hbm.at[ib[b].at[0]], gb[b], gsem[b]).wait()
        pltpu.make_async_copy(eidx_hbm.at[wid, j], ib[nb], isem[nb]).wait()
        pltpu.async_copy(x_hbm.at[ib[nb].at[0]], gb[nb], gsem[nb])
        pltpu.sync_copy(gb[b], acc.at[ib[b].at[1]], add=True)
        if fire_idx:
            pltpu.async_copy(eidx_hbm.at[wid, j + 2], ib[b], isem[b])

    def _pair(i, _):
        j = i * 2
        _half_step(j, 0, True)
        _half_step(j + 1, 1, True)
        return 0

    lax.fori_loop(0, NCH // 2 - 1, _pair, 0)
    _half_step(NCH - 2, 0, False)
    pltpu.make_async_copy(x_hbm.at[i1.at[0]], g1, gsem1).wait()
    pltpu.sync_copy(g1, acc.at[i1.at[1]], add=True)

    plsc.subcore_barrier()
    pltpu.sync_copy(acc.at[pl.ds(s * SW, SW)],
                    out_hbm.at[c].at[pl.ds(s * SW, SW)])

    @pl.when(s == NS - 1)
    def _write_tail():
        pltpu.sync_copy(acc.at[pl.ds(NS * SW, TAIL)],
                        out_hbm.at[c].at[pl.ds(NS * SW, TAIL)])


def _mlp_body(p_ref, w1_ref, b1_ref, g_ref, be_ref, w2_ref, b2_ref, o_ref):
    xu = p_ref[0] + p_ref[1]
    h = lax.dot_general(xu, w1_ref[...], (((1,), (1,)), ((), ())),
                        preferred_element_type=jnp.float32)
    h = h + b1_ref[...]
    mean = jnp.mean(h, axis=0, keepdims=True)
    d = h - mean
    var = jnp.mean(d * d, axis=0, keepdims=True)
    h = g_ref[...] * d * lax.rsqrt(var + BN_EPS) + be_ref[...]
    h = jnp.maximum(h, 0.0)
    o_ref[...] = lax.dot_general(h, w2_ref[...], (((1,), (1,)), ((), ())),
                                 preferred_element_type=jnp.float32) + b2_ref[...]


@jax.jit
def kernel(x, edge_index, W1, b1, gamma, beta, W2, b2):
    ei = edge_index.astype(jnp.int32)
    eidx = jnp.stack(
        [ei[1].reshape(NW, NCH, CH), ei[0].reshape(NW, NCH, CH)], axis=2)
    parts = _sc_agg(x, eidx)
    return pl.pallas_call(
        _mlp_body,
        out_shape=jax.ShapeDtypeStruct((N, D), jnp.float32),
    )(parts, W1, b1.reshape(1, D), gamma.reshape(1, D),
      beta.reshape(1, D), W2, b2.reshape(1, D))
